# baseline (speedup 1.0000x reference)
import jax
import jax.numpy as jnp
from jax.experimental import pallas as pl

N = 10000
X_DIM = 128
H_DIM = 128
G_DIM = 128
BLK = 1000


def _fused_mlp_kernel(x_ref, t_ref, wphi_ref, bphi_ref, wg_ref, bg_ref,
                      wg2_ref, bg2_ref, wh_ref, wp_ref, by_ref,
                      phi_ref, y2_ref):
    i = pl.program_id(0)
    x = x_ref[...]
    phi = jnp.dot(x, wphi_ref[...], preferred_element_type=jnp.float32)
    phi = phi + bphi_ref[...]
    phi_ref[...] = phi

    h = t_ref[...] * phi
    h = jnp.dot(h, wg_ref[...], preferred_element_type=jnp.float32) + bg_ref[...]
    h = jnp.maximum(h, 0.0)
    h = jnp.dot(h, wg2_ref[...], preferred_element_type=jnp.float32) + bg2_ref[...]
    h = jnp.maximum(h, 0.0)

    y2 = (jnp.dot(h, wh_ref[...], preferred_element_type=jnp.float32)
          + jnp.dot(phi, wp_ref[...], preferred_element_type=jnp.float32)
          + by_ref[...])
    y2_ref[pl.ds(i * BLK, BLK), :] = y2


def kernel(features, treatments, edge_index, W_phi, b_phi, W_g, b_g,
           W_g2, b_g2, W_t01, b_t01, W_t11, b_t11):
    del edge_index
    t2 = treatments[:, None]
    wh = jnp.concatenate([W_t01[H_DIM:], W_t11[H_DIM:]], axis=1)
    wp = jnp.concatenate([jnp.zeros_like(W_t11[:H_DIM]), W_t11[:H_DIM]],
                         axis=1)
    by = jnp.stack([b_t01[0], b_t11[0]])[None, :]

    grid = (N // BLK,)
    row_spec = pl.BlockSpec((BLK, X_DIM), lambda i: (i, 0))
    t_spec = pl.BlockSpec((BLK, 1), lambda i: (i, 0))

    def full(shape):
        return pl.BlockSpec(shape, lambda i: (0,) * len(shape))

    phi_x, y2 = pl.pallas_call(
        _fused_mlp_kernel,
        grid=grid,
        in_specs=[
            row_spec,
            t_spec,
            full((X_DIM, H_DIM)),
            full((1, H_DIM)),
            full((H_DIM, G_DIM)),
            full((1, G_DIM)),
            full((G_DIM, G_DIM)),
            full((1, G_DIM)),
            full((G_DIM, 2)),
            full((H_DIM, 2)),
            full((1, 2)),
        ],
        out_specs=[row_spec, full((N, 2))],
        out_shape=[
            jax.ShapeDtypeStruct((N, H_DIM), jnp.float32),
            jax.ShapeDtypeStruct((N, 2), jnp.float32),
        ],
    )(features, t2, W_phi, b_phi[None, :], W_g, b_g[None, :],
      W_g2, b_g2[None, :], wh, wp, by)

    return (y2[:, 1], y2[:, 0], phi_x)


# speedup vs baseline: 1.0157x; 1.0157x over previous
import jax
import jax.numpy as jnp
from jax.experimental import pallas as pl

N = 10000
X_DIM = 128
H_DIM = 128
G_DIM = 128
BLK = 1000


def _fused_mlp_kernel(x_ref, t_ref, wphi_ref, bphi_ref, wg_ref, bg_ref,
                      wg2_ref, bg2_ref, wh_ref, wp_ref, by_ref,
                      phi_ref, y2_ref):
    i = pl.program_id(0)
    x = x_ref[...]
    phi = jnp.dot(x, wphi_ref[...], preferred_element_type=jnp.float32)
    phi = phi + bphi_ref[...]
    phi_ref[...] = phi

    h = t_ref[...] * phi
    h = jnp.dot(h, wg_ref[...], preferred_element_type=jnp.float32) + bg_ref[...]
    h = jnp.maximum(h, 0.0)
    h = jnp.dot(h, wg2_ref[...], preferred_element_type=jnp.float32) + bg2_ref[...]
    h = jnp.maximum(h, 0.0)

    y128 = (jnp.dot(h, wh_ref[...], preferred_element_type=jnp.float32)
            + jnp.dot(phi, wp_ref[...], preferred_element_type=jnp.float32)
            + by_ref[...])
    y2_ref[pl.ds(i * BLK, BLK), :] = y128[:, :2]


def kernel(features, treatments, edge_index, W_phi, b_phi, W_g, b_g,
           W_g2, b_g2, W_t01, b_t01, W_t11, b_t11):
    del edge_index
    t2 = treatments[:, None]
    pad = jnp.zeros((G_DIM, 126), jnp.float32)
    wh = jnp.concatenate([W_t01[H_DIM:], W_t11[H_DIM:], pad], axis=1)
    wp = jnp.concatenate([jnp.zeros_like(W_t11[:H_DIM]), W_t11[:H_DIM],
                          pad], axis=1)
    by = jnp.concatenate([b_t01, b_t11, jnp.zeros((126,), jnp.float32)]
                         )[None, :]

    grid = (N // BLK,)
    row_spec = pl.BlockSpec((BLK, X_DIM), lambda i: (i, 0))
    t_spec = pl.BlockSpec((BLK, 1), lambda i: (i, 0))

    def full(shape):
        return pl.BlockSpec(shape, lambda i: (0,) * len(shape))

    phi_x, y2 = pl.pallas_call(
        _fused_mlp_kernel,
        grid=grid,
        in_specs=[
            row_spec,
            t_spec,
            full((X_DIM, H_DIM)),
            full((1, H_DIM)),
            full((H_DIM, G_DIM)),
            full((1, G_DIM)),
            full((G_DIM, G_DIM)),
            full((1, G_DIM)),
            full((G_DIM, H_DIM)),
            full((H_DIM, H_DIM)),
            full((1, H_DIM)),
        ],
        out_specs=[row_spec, full((N, 2))],
        out_shape=[
            jax.ShapeDtypeStruct((N, H_DIM), jnp.float32),
            jax.ShapeDtypeStruct((N, 2), jnp.float32),
        ],
    )(features, t2, W_phi, b_phi[None, :], W_g, b_g[None, :],
      W_g2, b_g2[None, :], wh, wp, by)

    return (y2[:, 1], y2[:, 0], phi_x)
